# baseline (speedup 1.0000x reference)
import functools

import jax
import jax.numpy as jnp
from jax.experimental import pallas as pl
from jax.experimental.pallas import tpu as pltpu


def _mm_body(adj_ref, w_ref, out_ref, *, bk: int):
    j = pl.program_id(1)

    @pl.when(j == 0)
    def _():
        out_ref[...] = jnp.zeros_like(out_ref)

    wk = w_ref[pl.ds(j * bk, bk), :]
    out_ref[...] += jnp.dot(adj_ref[...], wk, preferred_element_type=jnp.float32)


def kernel(adj, weight):
    m, k = adj.shape
    k2, n = weight.shape
    assert k == k2
    bm = 512
    bk = 2048
    grid = (m // bm, k // bk)
    return pl.pallas_call(
        functools.partial(_mm_body, bk=bk),
        grid=grid,
        in_specs=[
            pl.BlockSpec((bm, bk), lambda i, j: (i, j)),
            pl.BlockSpec((k2, n), lambda i, j: (0, 0)),
        ],
        out_specs=pl.BlockSpec((bm, n), lambda i, j: (i, 0)),
        out_shape=jax.ShapeDtypeStruct((m, n), jnp.float32),
        compiler_params=pltpu.CompilerParams(
            dimension_semantics=("parallel", "arbitrary"),
        ),
    )(adj, weight)
